# Initial kernel scaffold; baseline (speedup 1.0000x reference)
import jax
import jax.numpy as jnp
from jax.experimental import pallas as pl


def kernel(X, A, W1, b1, ln1_g, ln1_b, W2, b2, ln2_g, ln2_b, mu, qw, qb, kw, kb, vw, vb, m1w, m1b, m2w, m2b, m3w, m3b):
    raise NotImplementedError("write your pallas kernel here")


# speedup vs baseline: 3.1089x; 3.1089x over previous
import functools

import jax
import jax.numpy as jnp
from jax.experimental import pallas as pl
from jax.experimental.pallas import tpu as pltpu

N = 4096
INP_D = 512
D_E = 256
M = 1024
KN = 10
KE = 10

_PREC = jax.lax.Precision.HIGHEST


def _dot(a, b, trans_b=False):
    dims = (((1,), (1 if trans_b else 0,)), ((), ()))
    return jax.lax.dot_general(a, b, dims, precision=_PREC,
                               preferred_element_type=jnp.float32)


def _leaky(x):
    return jnp.where(x >= 0, x, 0.01 * x)


def _layer_norm(x, g, b, eps=1e-5):
    mu = jnp.mean(x, axis=-1, keepdims=True)
    var = jnp.mean((x - mu) ** 2, axis=-1, keepdims=True)
    return (x - mu) / jnp.sqrt(var + eps) * g + b


def _topk_mask(P, k):
    C = P.shape[1]
    col = jax.lax.broadcasted_iota(jnp.int32, P.shape, 1)
    sel = jnp.zeros(P.shape, jnp.bool_)
    for _ in range(k):
        cur = jnp.where(sel, -1.0, P)
        m = jnp.max(cur, axis=1, keepdims=True)
        idx = jnp.min(jnp.where(cur == m, col, C), axis=1, keepdims=True)
        sel = jnp.logical_or(sel, col == idx)
    return sel.astype(P.dtype)


def _prep_kernel(a_ref, x_ref, w1_ref, dis_ref, zp1_ref):
    deg = jnp.sum(a_ref[...], axis=1, keepdims=True)
    dis = jnp.where(deg > 0, jax.lax.rsqrt(deg), 0.0)
    dis_ref[...] = dis
    zp1_ref[...] = dis * _dot(x_ref[...], w1_ref[...])


def _gcn1_kernel(a_ref, zp_ref, dis_ref, b_ref, g_ref, bb_ref, w2_ref,
                 out_ref, acc_ref, *, nj):
    j = pl.program_id(1)

    @pl.when(j == 0)
    def _():
        acc_ref[...] = jnp.zeros_like(acc_ref)

    acc_ref[...] += _dot(a_ref[...], zp_ref[...])

    @pl.when(j == nj - 1)
    def _():
        y = dis_ref[...] * acc_ref[...] + b_ref[...]
        y = _leaky(_layer_norm(y, g_ref[...], bb_ref[...]))
        out_ref[...] = dis_ref[...] * _dot(y, w2_ref[...])


def _gcn2_kernel(a_ref, zp_ref, dis_ref, b_ref, g_ref, bb_ref,
                 kw_ref, kb_ref, vw_ref, vb_ref, qw_ref, qb_ref,
                 k1_ref, v1_ref, q2_ref, acc_ref, *, nj):
    j = pl.program_id(1)

    @pl.when(j == 0)
    def _():
        acc_ref[...] = jnp.zeros_like(acc_ref)

    acc_ref[...] += _dot(a_ref[...], zp_ref[...])

    @pl.when(j == nj - 1)
    def _():
        y = dis_ref[...] * acc_ref[...] + b_ref[...]
        xv = _layer_norm(y, g_ref[...], bb_ref[...])
        k1_ref[...] = _dot(xv, kw_ref[...]) + kb_ref[...]
        v1_ref[...] = _dot(xv, vw_ref[...]) + vb_ref[...]
        q2_ref[...] = _dot(xv, qw_ref[...]) + qb_ref[...]


def _attn1_kernel(mu_ref, qw_ref, qb_ref, k1_ref, v1_ref,
                  m1wa_ref, m1wb_ref, m1b_ref, m2w_ref, m2b_ref,
                  m3w_ref, m3b_ref, kw_ref, kb_ref,
                  xe_ref, k2_ref):
    mu = mu_ref[...]
    q1 = _dot(mu, qw_ref[...]) + qb_ref[...]
    logits = _dot(q1, k1_ref[...], trans_b=True) * (1.0 / 16.0)
    mx = jnp.max(logits, axis=1, keepdims=True)
    p = jnp.exp(logits - mx)
    e2v = p / jnp.sum(p, axis=1, keepdims=True)
    mask = _topk_mask(e2v, KN)
    agg = _dot(mask, v1_ref[...])
    h = _leaky(_dot(mu, m1wa_ref[...]) + _dot(agg, m1wb_ref[...]) + m1b_ref[...])
    h = _leaky(_dot(h, m2w_ref[...]) + m2b_ref[...])
    xe = _dot(h, m3w_ref[...]) + m3b_ref[...]
    xe_ref[...] = xe
    k2_ref[...] = _dot(xe, kw_ref[...]) + kb_ref[...]


def _attn2_kernel(q2_ref, k2_ref, v2e_ref, h_ref):
    logits = _dot(q2_ref[...], k2_ref[...], trans_b=True) * (1.0 / 16.0)
    mx = jnp.max(logits, axis=1, keepdims=True)
    p = jnp.exp(logits - mx)
    v2e = p / jnp.sum(p, axis=1, keepdims=True)
    v2e_ref[...] = v2e
    h_ref[...] = _topk_mask(v2e, KE)


def kernel(X, A, W1, b1, ln1_g, ln1_b, W2, b2, ln2_g, ln2_b, mu, qw, qb,
           kw, kb, vw, vb, m1w, m1b, m2w, m2b, m3w, m3b):
    f32 = jnp.float32

    def row2d(v):
        return v.reshape(1, -1)

    BR1 = 256
    dis, zp1 = pl.pallas_call(
        _prep_kernel,
        grid=(N // BR1,),
        in_specs=[
            pl.BlockSpec((BR1, N), lambda i: (i, 0)),
            pl.BlockSpec((BR1, INP_D), lambda i: (i, 0)),
            pl.BlockSpec((INP_D, D_E), lambda i: (0, 0)),
        ],
        out_specs=[
            pl.BlockSpec((BR1, 1), lambda i: (i, 0)),
            pl.BlockSpec((BR1, D_E), lambda i: (i, 0)),
        ],
        out_shape=[
            jax.ShapeDtypeStruct((N, 1), f32),
            jax.ShapeDtypeStruct((N, D_E), f32),
        ],
        compiler_params=pltpu.CompilerParams(
            dimension_semantics=("arbitrary",)),
    )(A, X, W1)

    BR, BC = 512, 1024
    NJ = N // BC
    zp2 = pl.pallas_call(
        functools.partial(_gcn1_kernel, nj=NJ),
        grid=(N // BR, NJ),
        in_specs=[
            pl.BlockSpec((BR, BC), lambda i, j: (i, j)),
            pl.BlockSpec((BC, D_E), lambda i, j: (j, 0)),
            pl.BlockSpec((BR, 1), lambda i, j: (i, 0)),
            pl.BlockSpec((1, D_E), lambda i, j: (0, 0)),
            pl.BlockSpec((1, D_E), lambda i, j: (0, 0)),
            pl.BlockSpec((1, D_E), lambda i, j: (0, 0)),
            pl.BlockSpec((D_E, D_E), lambda i, j: (0, 0)),
        ],
        out_specs=pl.BlockSpec((BR, D_E), lambda i, j: (i, 0)),
        out_shape=jax.ShapeDtypeStruct((N, D_E), f32),
        scratch_shapes=[pltpu.VMEM((BR, D_E), f32)],
        compiler_params=pltpu.CompilerParams(
            dimension_semantics=("parallel", "arbitrary")),
    )(A, zp1, dis, row2d(b1), row2d(ln1_g), row2d(ln1_b), W2)

    k1, v1, q2 = pl.pallas_call(
        functools.partial(_gcn2_kernel, nj=NJ),
        grid=(N // BR, NJ),
        in_specs=[
            pl.BlockSpec((BR, BC), lambda i, j: (i, j)),
            pl.BlockSpec((BC, D_E), lambda i, j: (j, 0)),
            pl.BlockSpec((BR, 1), lambda i, j: (i, 0)),
            pl.BlockSpec((1, D_E), lambda i, j: (0, 0)),
            pl.BlockSpec((1, D_E), lambda i, j: (0, 0)),
            pl.BlockSpec((1, D_E), lambda i, j: (0, 0)),
            pl.BlockSpec((D_E, D_E), lambda i, j: (0, 0)),
            pl.BlockSpec((1, D_E), lambda i, j: (0, 0)),
            pl.BlockSpec((D_E, D_E), lambda i, j: (0, 0)),
            pl.BlockSpec((1, D_E), lambda i, j: (0, 0)),
            pl.BlockSpec((D_E, D_E), lambda i, j: (0, 0)),
            pl.BlockSpec((1, D_E), lambda i, j: (0, 0)),
        ],
        out_specs=[
            pl.BlockSpec((BR, D_E), lambda i, j: (i, 0)),
            pl.BlockSpec((BR, D_E), lambda i, j: (i, 0)),
            pl.BlockSpec((BR, D_E), lambda i, j: (i, 0)),
        ],
        out_shape=[
            jax.ShapeDtypeStruct((N, D_E), f32),
            jax.ShapeDtypeStruct((N, D_E), f32),
            jax.ShapeDtypeStruct((N, D_E), f32),
        ],
        scratch_shapes=[pltpu.VMEM((BR, D_E), f32)],
        compiler_params=pltpu.CompilerParams(
            dimension_semantics=("parallel", "arbitrary")),
    )(A, zp2, dis, row2d(b2), row2d(ln2_g), row2d(ln2_b),
      kw, row2d(kb), vw, row2d(vb), qw, row2d(qb))

    BM = 256
    m1wa, m1wb = m1w[:D_E], m1w[D_E:]
    xe, k2 = pl.pallas_call(
        _attn1_kernel,
        grid=(M // BM,),
        in_specs=[
            pl.BlockSpec((BM, D_E), lambda i: (i, 0)),
            pl.BlockSpec((D_E, D_E), lambda i: (0, 0)),
            pl.BlockSpec((1, D_E), lambda i: (0, 0)),
            pl.BlockSpec((N, D_E), lambda i: (0, 0)),
            pl.BlockSpec((N, D_E), lambda i: (0, 0)),
            pl.BlockSpec((D_E, 2 * D_E), lambda i: (0, 0)),
            pl.BlockSpec((D_E, 2 * D_E), lambda i: (0, 0)),
            pl.BlockSpec((1, 2 * D_E), lambda i: (0, 0)),
            pl.BlockSpec((2 * D_E, D_E), lambda i: (0, 0)),
            pl.BlockSpec((1, D_E), lambda i: (0, 0)),
            pl.BlockSpec((D_E, D_E), lambda i: (0, 0)),
            pl.BlockSpec((1, D_E), lambda i: (0, 0)),
            pl.BlockSpec((D_E, D_E), lambda i: (0, 0)),
            pl.BlockSpec((1, D_E), lambda i: (0, 0)),
        ],
        out_specs=[
            pl.BlockSpec((BM, D_E), lambda i: (i, 0)),
            pl.BlockSpec((BM, D_E), lambda i: (i, 0)),
        ],
        out_shape=[
            jax.ShapeDtypeStruct((M, D_E), f32),
            jax.ShapeDtypeStruct((M, D_E), f32),
        ],
        compiler_params=pltpu.CompilerParams(
            dimension_semantics=("arbitrary",)),
    )(mu, qw, row2d(qb), k1, v1, m1wa, m1wb, row2d(m1b),
      m2w, row2d(m2b), m3w, row2d(m3b), kw, row2d(kb))

    BQ = 512
    v2e, H = pl.pallas_call(
        _attn2_kernel,
        grid=(N // BQ,),
        in_specs=[
            pl.BlockSpec((BQ, D_E), lambda i: (i, 0)),
            pl.BlockSpec((M, D_E), lambda i: (0, 0)),
        ],
        out_specs=[
            pl.BlockSpec((BQ, M), lambda i: (i, 0)),
            pl.BlockSpec((BQ, M), lambda i: (i, 0)),
        ],
        out_shape=[
            jax.ShapeDtypeStruct((N, M), f32),
            jax.ShapeDtypeStruct((N, M), f32),
        ],
        compiler_params=pltpu.CompilerParams(
            dimension_semantics=("arbitrary",)),
    )(q2, k2)

    return (H, xe, v2e)
